# Initial kernel scaffold; baseline (speedup 1.0000x reference)
import jax
import jax.numpy as jnp
from jax.experimental import pallas as pl


def kernel(x, emb, W1, b1, W2, b2):
    raise NotImplementedError("write your pallas kernel here")


# speedup vs baseline: 12.4604x; 12.4604x over previous
import functools

import jax
import jax.numpy as jnp
from jax import lax
from jax.experimental import pallas as pl
from jax.experimental.pallas import tpu as pltpu
from jax.experimental.pallas import tpu_sc as plsc

NR_HASH = 1000000
EMBED_DIM = 64
HIDDEN = 128
NUM_CLASSES = 10
NB_KEYS = 3
BATCH = 16384
HIST = 50

IDX_PER_B = NB_KEYS * HIST
NC, NS = 2, 16
NW = NC * NS
B_PER_W = BATCH // NW
CB = 4
ROWS_PER_CHUNK = CB * IDX_PER_B
DMAS_PER_CHUNK = 8
IDX_PER_DMA = ROWS_PER_CHUNK // DMAS_PER_CHUNK
CHUNKS = B_PER_W // CB


def _sc_pool(xt, emb):
    mesh = plsc.VectorSubcoreMesh(core_axis_name="c", subcore_axis_name="s")

    def body(xt_hbm, emb_hbm, out_hbm, idx_v, rows_v, out_v, sem):
        wid = lax.axis_index("s") * NC + lax.axis_index("c")
        base_b = wid * B_PER_W

        def chunk(c, _):
            b0 = base_b + c * CB
            pltpu.sync_copy(xt_hbm.at[pl.ds(b0 * 2, DMAS_PER_CHUNK)], idx_v)
            copies = []
            for j in range(DMAS_PER_CHUNK):
                copies.append(
                    pltpu.async_copy(
                        emb_hbm.at[idx_v.at[j]],
                        rows_v.at[pl.ds(j * IDX_PER_DMA, IDX_PER_DMA)],
                        sem,
                    )
                )
            for cp in copies:
                cp.wait()
            for bb in range(CB):
                def rbody(r, accs):
                    a0, a1, a2, a3 = accs
                    row = bb * IDX_PER_B + r
                    a0 = a0 + rows_v[row, pl.ds(0, 16)]
                    a1 = a1 + rows_v[row, pl.ds(16, 16)]
                    a2 = a2 + rows_v[row, pl.ds(32, 16)]
                    a3 = a3 + rows_v[row, pl.ds(48, 16)]
                    return (a0, a1, a2, a3)

                z = jnp.zeros((16,), jnp.float32)
                a0, a1, a2, a3 = lax.fori_loop(0, IDX_PER_B, rbody, (z, z, z, z))
                out_v[bb, pl.ds(0, 16)] = a0
                out_v[bb, pl.ds(16, 16)] = a1
                out_v[bb, pl.ds(32, 16)] = a2
                out_v[bb, pl.ds(48, 16)] = a3
            pltpu.sync_copy(out_v, out_hbm.at[pl.ds(b0, CB)])
            return 0

        lax.fori_loop(0, CHUNKS, chunk, 0)

    return pl.kernel(
        body,
        out_type=jax.ShapeDtypeStruct((BATCH, EMBED_DIM), jnp.float32),
        mesh=mesh,
        scratch_types=[
            pltpu.VMEM((DMAS_PER_CHUNK, IDX_PER_DMA), jnp.int32),
            pltpu.VMEM((ROWS_PER_CHUNK, EMBED_DIM), jnp.float32),
            pltpu.VMEM((CB, EMBED_DIM), jnp.float32),
            pltpu.SemaphoreType.DMA,
        ],
        compiler_params=pltpu.CompilerParams(use_tc_tiling_on_sc=False),
    )(xt, emb)


def _mlp_body(p_ref, w1_ref, b1_ref, w2_ref, b2_ref, o_ref):
    h = jnp.dot(p_ref[...], w1_ref[...], preferred_element_type=jnp.float32)
    h = jnp.maximum(h + b1_ref[...], 0.0)
    o_ref[...] = (
        jnp.dot(h, w2_ref[...], preferred_element_type=jnp.float32) + b2_ref[...]
    )


def _mlp(pooled, W1s, b1, W2, b2):
    TM = 2048
    grid = (BATCH // TM,)
    return pl.pallas_call(
        _mlp_body,
        grid=grid,
        in_specs=[
            pl.BlockSpec((TM, EMBED_DIM), lambda i: (i, 0)),
            pl.BlockSpec((EMBED_DIM, HIDDEN), lambda i: (0, 0)),
            pl.BlockSpec((1, HIDDEN), lambda i: (0, 0)),
            pl.BlockSpec((HIDDEN, NUM_CLASSES), lambda i: (0, 0)),
            pl.BlockSpec((1, NUM_CLASSES), lambda i: (0, 0)),
        ],
        out_specs=pl.BlockSpec((TM, NUM_CLASSES), lambda i: (i, 0)),
        out_shape=jax.ShapeDtypeStruct((BATCH, NUM_CLASSES), jnp.float32),
    )(pooled, W1s, b1, W2, b2)


@jax.jit
def kernel(x, emb, W1, b1, W2, b2):
    xt = jnp.transpose(x, (1, 0, 2)).reshape(BATCH * 2, IDX_PER_B // 2)
    pooled = _sc_pool(xt, emb)
    W1s = W1 * (1.0 / HIST)
    out = _mlp(pooled, W1s, b1.reshape(1, HIDDEN), W2, b2.reshape(1, NUM_CLASSES))
    return out


# speedup vs baseline: 16.7390x; 1.3434x over previous
import functools

import jax
import jax.numpy as jnp
from jax import lax
from jax.experimental import pallas as pl
from jax.experimental.pallas import tpu as pltpu
from jax.experimental.pallas import tpu_sc as plsc

NR_HASH = 1000000
EMBED_DIM = 64
HIDDEN = 128
NUM_CLASSES = 10
NB_KEYS = 3
BATCH = 16384
HIST = 50

IDX_PER_B = NB_KEYS * HIST
NC, NS = 2, 16
NW = NC * NS
B_PER_W = BATCH // NW
CB = 4
ROWS_PER_CHUNK = CB * IDX_PER_B
DMAS_PER_CHUNK = 8
IDX_PER_DMA = ROWS_PER_CHUNK // DMAS_PER_CHUNK
CHUNKS = B_PER_W // CB


def _sc_pool(xt, emb):
    mesh = plsc.VectorSubcoreMesh(core_axis_name="c", subcore_axis_name="s")
    UNROLL = 6
    RITERS = IDX_PER_B // UNROLL

    def body(xt_hbm, emb_hbm, out_hbm, idx_v, rows_v, out_v, sem0, sem1):
        wid = lax.axis_index("s") * NC + lax.axis_index("c")
        base_b = wid * B_PER_W
        sems = (sem0, sem1)

        def issue(c, p):
            b0 = base_b + c * CB
            pltpu.sync_copy(xt_hbm.at[pl.ds(b0 * 2, DMAS_PER_CHUNK)], idx_v.at[p])
            for j in range(DMAS_PER_CHUNK):
                pltpu.async_copy(
                    emb_hbm.at[idx_v.at[p].at[j]],
                    rows_v.at[p].at[pl.ds(j * IDX_PER_DMA, IDX_PER_DMA)],
                    sems[p],
                )

        def drain(p):
            for j in range(DMAS_PER_CHUNK):
                pltpu.make_async_copy(
                    emb_hbm.at[idx_v.at[p].at[j]],
                    rows_v.at[p].at[pl.ds(j * IDX_PER_DMA, IDX_PER_DMA)],
                    sems[p],
                ).wait()

        def reduce_store(c, p):
            rows = rows_v.at[p]
            for bb in range(CB):
                def rbody(r, accs):
                    accs = list(accs)
                    for u in range(UNROLL):
                        row = bb * IDX_PER_B + r * UNROLL + u
                        for d in range(4):
                            s = d + (u % 2) * 4
                            accs[s] = accs[s] + rows[row, pl.ds(d * 16, 16)]
                    return tuple(accs)

                z = jnp.zeros((16,), jnp.float32)
                accs = lax.fori_loop(0, RITERS, rbody, (z,) * 8)
                for d in range(4):
                    out_v[bb, pl.ds(d * 16, 16)] = accs[d] + accs[d + 4]
            pltpu.sync_copy(out_v, out_hbm.at[pl.ds(base_b + c * CB, CB)])

        issue(0, 0)
        issue(1, 1)

        def outer(g, _):
            for p in range(2):
                c = g * 2 + p
                drain(p)
                reduce_store(c, p)

                @pl.when(c + 2 < CHUNKS)
                def _():
                    issue(c + 2, p)
            return 0

        lax.fori_loop(0, CHUNKS // 2, outer, 0)

    return pl.kernel(
        body,
        out_type=jax.ShapeDtypeStruct((BATCH, EMBED_DIM), jnp.float32),
        mesh=mesh,
        scratch_types=[
            pltpu.VMEM((2, DMAS_PER_CHUNK, IDX_PER_DMA), jnp.int32),
            pltpu.VMEM((2, ROWS_PER_CHUNK, EMBED_DIM), jnp.float32),
            pltpu.VMEM((CB, EMBED_DIM), jnp.float32),
            pltpu.SemaphoreType.DMA,
            pltpu.SemaphoreType.DMA,
        ],
        compiler_params=pltpu.CompilerParams(use_tc_tiling_on_sc=False),
    )(xt, emb)


def _mlp_body(p_ref, w1_ref, b1_ref, w2_ref, b2_ref, o_ref):
    h = jnp.dot(p_ref[...], w1_ref[...], preferred_element_type=jnp.float32)
    h = jnp.maximum(h + b1_ref[...], 0.0)
    o_ref[...] = (
        jnp.dot(h, w2_ref[...], preferred_element_type=jnp.float32) + b2_ref[...]
    )


def _mlp(pooled, W1s, b1, W2, b2):
    TM = 2048
    grid = (BATCH // TM,)
    return pl.pallas_call(
        _mlp_body,
        grid=grid,
        in_specs=[
            pl.BlockSpec((TM, EMBED_DIM), lambda i: (i, 0)),
            pl.BlockSpec((EMBED_DIM, HIDDEN), lambda i: (0, 0)),
            pl.BlockSpec((1, HIDDEN), lambda i: (0, 0)),
            pl.BlockSpec((HIDDEN, NUM_CLASSES), lambda i: (0, 0)),
            pl.BlockSpec((1, NUM_CLASSES), lambda i: (0, 0)),
        ],
        out_specs=pl.BlockSpec((TM, NUM_CLASSES), lambda i: (i, 0)),
        out_shape=jax.ShapeDtypeStruct((BATCH, NUM_CLASSES), jnp.float32),
    )(pooled, W1s, b1, W2, b2)


@jax.jit
def kernel(x, emb, W1, b1, W2, b2):
    xt = jnp.transpose(x, (1, 0, 2)).reshape(BATCH * 2, IDX_PER_B // 2)
    pooled = _sc_pool(xt, emb)
    W1s = W1 * (1.0 / HIST)
    out = _mlp(pooled, W1s, b1.reshape(1, HIDDEN), W2, b2.reshape(1, NUM_CLASSES))
    return out
